# baseline (speedup 1.0000x reference)
import functools

import jax
import jax.numpy as jnp
from jax import lax
from jax.experimental import pallas as pl
from jax.experimental.pallas import tpu as pltpu
from jax.experimental.pallas import tpu_sc as plsc

V = 1000
V3 = 3 * V
VP = 3008
D = 128
B = 1024
P = 64
C = 3
L = 16

NC = 2
NS = 16
NW = NC * NS

ND = 8
DC = D // ND
NB = NW // ND
BC = B // NB
G = 8
NG = BC // G
SL = DC + 1

_mesh = plsc.VectorSubcoreMesh(core_axis_name="c", subcore_axis_name="s")


@functools.partial(
    pl.kernel,
    out_type=jax.ShapeDtypeStruct((B, D, P), jnp.float32),
    mesh=_mesh,
    compiler_params=pltpu.CompilerParams(
        use_tc_tiling_on_sc=False, needs_layout_passes=False),
    scratch_types=[
        pltpu.VMEM((VP, SL), jnp.float32),
        pltpu.VMEM((DC // 2, VP), jnp.int32),
        pltpu.VMEM((2, G, P * C), jnp.int32),
        pltpu.VMEM((2, G, DC, P), jnp.float32),
        pltpu.SemaphoreType.DMA((2,)),
        pltpu.SemaphoreType.DMA((2,)),
    ],
)
def _bow_kernel(emb_hbm, idx_hbm, out_hbm, slab_v, tab_v, idx_v, out_v,
                idx_sem, out_sem):
    wid = lax.axis_index("s") * NC + lax.axis_index("c")
    d0 = (wid % ND) * DC
    b0 = (wid // ND) * BC

    def idx_copy(g, k):
        return pltpu.make_async_copy(
            idx_hbm.at[pl.ds(b0 + g * G, G)], idx_v.at[k], idx_sem.at[k])

    def out_copy(g, k):
        return pltpu.make_async_copy(
            out_v.at[k],
            out_hbm.at[pl.ds(b0 + g * G, G), pl.ds(d0, DC)],
            out_sem.at[k])

    idx_copy(0, 0).start()
    idx_copy(1, 1).start()

    pltpu.sync_copy(emb_hbm.at[:, pl.ds(d0, DC)],
                    slab_v.at[pl.ds(0, V3), pl.ds(0, DC)])
    lane = lax.iota(jnp.int32, 16)

    def build(vg, carry):
        vrow = vg * L + lane
        for j in range(DC // 2):
            a = plsc.load_gather(slab_v, [vrow, jnp.full((16,), 2 * j,
                                                         jnp.int32)])
            b = plsc.load_gather(slab_v, [vrow, jnp.full((16,), 2 * j + 1,
                                                         jnp.int32)])
            pk = plsc.pack(a, b, format=plsc.PackFormat.INTERLEAVED)
            tab_v[j, pl.ds(vg * L, L)] = plsc.bitcast(pk, jnp.int32)
        return carry

    lax.fori_loop(0, VP // L, build, None)

    lane3 = lane * 3

    def outer(g0, carry):
        for k in range(2):
            g = g0 + k
            idx_copy(g, k).wait()
            @pl.when(g0 >= 2)
            def _():
                out_copy(g - 2, k).wait()

            def img(i, c2):
                for pg in range(P // L):
                    idx_c = []
                    for c in range(C):
                        iv = plsc.load_gather(
                            idx_v, [jnp.full((16,), k, jnp.int32),
                                    jnp.full((16,), i, jnp.int32),
                                    lane3 + (pg * L * C + c)])
                        if c:
                            iv = iv + c * V
                        idx_c.append(iv)
                    for j in range(DC // 2):
                        row = jnp.full((16,), j, jnp.int32)
                        s = plsc.bitcast(
                            plsc.load_gather(tab_v, [row, idx_c[0]]),
                            jnp.bfloat16)
                        s = s + plsc.bitcast(
                            plsc.load_gather(tab_v, [row, idx_c[1]]),
                            jnp.bfloat16)
                        s = s + plsc.bitcast(
                            plsc.load_gather(tab_v, [row, idx_c[2]]),
                            jnp.bfloat16)
                        lo, hi = plsc.unpack(
                            s, format=plsc.PackFormat.INTERLEAVED)
                        out_v[k, i, 2 * j, pl.ds(pg * L, L)] = lo
                        out_v[k, i, 2 * j + 1, pl.ds(pg * L, L)] = hi
                return c2

            lax.fori_loop(0, G, img, None)
            out_copy(g, k).start()
            @pl.when(g + 2 < NG)
            def _():
                idx_copy(g + 2, k).start()
        return carry

    lax.fori_loop(0, NG // 2, lambda t, c: outer(t * 2, c), None)

    out_copy(NG - 2, 0).wait()
    out_copy(NG - 1, 1).wait()


def kernel(inputs, embedding):
    b, h, w, c = inputs.shape
    idx = inputs.reshape(b, h * w * c).astype(jnp.int32)
    out = _bow_kernel(embedding, idx)
    return out.reshape(b, D, h, w)


# speedup vs baseline: 1.7478x; 1.7478x over previous
import functools

import jax
import jax.numpy as jnp
from jax import lax
from jax.experimental import pallas as pl
from jax.experimental.pallas import tpu as pltpu
from jax.experimental.pallas import tpu_sc as plsc

V = 1000
V3 = 3 * V
D = 128
B = 1024
P = 64
C = 3
L = 16

NC = 2
NS = 16
NW = NC * NS

ND = 4
DW = D // ND
NB = NW // ND
BC = B // NB
G = 8
NG = BC // G

_mesh = plsc.VectorSubcoreMesh(core_axis_name="c", subcore_axis_name="s")


@functools.partial(
    pl.kernel,
    out_type=jax.ShapeDtypeStruct((B, P, D), jnp.float32),
    mesh=_mesh,
    compiler_params=pltpu.CompilerParams(
        use_tc_tiling_on_sc=False, needs_layout_passes=False),
    scratch_types=[
        pltpu.VMEM((V3, L), jnp.int32),
        pltpu.VMEM((2, G, P * C), jnp.int32),
        pltpu.VMEM((2, G, P, DW), jnp.float32),
        pltpu.SemaphoreType.DMA((2,)),
        pltpu.SemaphoreType.DMA((2,)),
    ],
)
def _bow_kernel(tab_hbm, idx_hbm, out_hbm, tab_v, idx_v, out_v,
                idx_sem, out_sem):
    wid = lax.axis_index("s") * NC + lax.axis_index("c")
    w = wid % ND
    b0 = (wid // ND) * BC

    def idx_copy(g, k):
        return pltpu.make_async_copy(
            idx_hbm.at[pl.ds(b0 + g * G, G)], idx_v.at[k], idx_sem.at[k])

    def out_copy(g, k):
        return pltpu.make_async_copy(
            out_v.at[k],
            out_hbm.at[pl.ds(b0 + g * G, G), :, pl.ds(w * DW, DW)],
            out_sem.at[k])

    idx_copy(0, 0).start()
    idx_copy(1, 1).start()
    pltpu.sync_copy(tab_hbm.at[w], tab_v)

    def outer(g0, carry):
        for k in range(2):
            g = g0 + k
            idx_copy(g, k).wait()
            @pl.when(g0 >= 2)
            def _():
                out_copy(g - 2, k).wait()

            def img(i, c2):
                for pg in range(P // L):
                    base = pg * L * C
                    vs = [idx_v[k, i, pl.ds(base + t * L, L)]
                          for t in range(C)]
                    for t in range(L):
                        w0, w1, w2 = 3 * t, 3 * t + 1, 3 * t + 2
                        s0 = vs[w0 // L][w0 % L]
                        s1 = vs[w1 // L][w1 % L] + V
                        s2 = vs[w2 // L][w2 % L] + 2 * V
                        r = plsc.bitcast(tab_v[s0, pl.ds(0, L)],
                                         jnp.bfloat16)
                        r = r + plsc.bitcast(tab_v[s1, pl.ds(0, L)],
                                             jnp.bfloat16)
                        r = r + plsc.bitcast(tab_v[s2, pl.ds(0, L)],
                                             jnp.bfloat16)
                        lo, hi = plsc.unpack(
                            r, format=plsc.PackFormat.INTERLEAVED)
                        p = pg * L + t
                        out_v[k, i, p, pl.ds(0, L)] = lo
                        out_v[k, i, p, pl.ds(L, L)] = hi
                return c2

            lax.fori_loop(0, G, img, None)
            out_copy(g, k).start()
            @pl.when(g + 2 < NG)
            def _():
                idx_copy(g + 2, k).start()
        return carry

    lax.fori_loop(0, NG // 2, lambda t, c: outer(t * 2, c), None)

    out_copy(NG - 2, 0).wait()
    out_copy(NG - 1, 1).wait()


def kernel(inputs, embedding):
    b, h, w, c = inputs.shape
    e4 = lax.bitcast_convert_type(
        embedding.reshape(V3, ND, 2, L).astype(jnp.bfloat16),
        jnp.uint16).astype(jnp.uint32)
    packed = lax.bitcast_convert_type(
        (e4[:, :, 1, :] << 16) | e4[:, :, 0, :], jnp.int32)
    packed = jnp.swapaxes(packed, 0, 1)
    idx = inputs.reshape(b, h * w * c).astype(jnp.int32)
    out = _bow_kernel(packed, idx)
    return jnp.transpose(out.reshape(b, h, w, D), (0, 3, 1, 2))


# speedup vs baseline: 2.4465x; 1.3997x over previous
import functools

import jax
import jax.numpy as jnp
from jax import lax
from jax.experimental import pallas as pl
from jax.experimental.pallas import tpu as pltpu
from jax.experimental.pallas import tpu_sc as plsc

V = 1000
V3 = 3 * V
VP = 3008
D = 128
B = 1024
P = 64
C = 3
L = 16

NC = 2
NS = 16
NW = NC * NS

BC = B // NW
G = 4
NG = BC // G

_mesh = plsc.VectorSubcoreMesh(core_axis_name="c", subcore_axis_name="s")


@functools.partial(
    pl.kernel,
    out_type=jax.ShapeDtypeStruct((B, P, D), jnp.float32),
    mesh=_mesh,
    compiler_params=pltpu.CompilerParams(
        use_tc_tiling_on_sc=False, needs_layout_passes=False),
    scratch_types=[
        pltpu.VMEM_SHARED((VP, D), jnp.bfloat16),
        pltpu.VMEM((2, G, P * C), jnp.int32),
        pltpu.VMEM((3, C, P), jnp.int32),
        pltpu.VMEM((3, P, D), jnp.bfloat16),
        pltpu.VMEM((2, G, P, D), jnp.float32),
        pltpu.SemaphoreType.DMA((2,)),
        pltpu.SemaphoreType.DMA((3,)),
        pltpu.SemaphoreType.DMA((2,)),
    ],
)
def _bow_kernel(tab_hbm, idx_hbm, out_hbm, tab_sh, idx_v, cidx_v, rows_v,
                out_v, idx_sem, rows_sem, out_sem):
    sid = lax.axis_index("s")
    wid = sid * NC + lax.axis_index("c")
    b0 = wid * BC

    def idx_copy(g, k):
        return pltpu.make_async_copy(
            idx_hbm.at[pl.ds(b0 + g * G, G)], idx_v.at[k], idx_sem.at[k])

    def out_copy(g, k):
        return pltpu.make_async_copy(
            out_v.at[k], out_hbm.at[pl.ds(b0 + g * G, G)], out_sem.at[k])

    def rows_copy(c, buf):
        return pltpu.make_async_copy(
            tab_sh.at[cidx_v.at[buf, c]], rows_v.at[buf],
            rows_sem.at[buf])

    idx_copy(0, 0).start()
    idx_copy(1, 1).start()
    rpt = VP // NS
    pltpu.sync_copy(tab_hbm.at[pl.ds(sid * rpt, rpt)],
                    tab_sh.at[pl.ds(sid * rpt, rpt)])
    plsc.subcore_barrier()

    lane3 = lax.iota(jnp.int32, 16) * C

    def build_and_fire0(k, i, buf):
        iref = idx_v.at[k, i]
        for c in range(C):
            for q in range(P // L):
                iv = plsc.load_gather(iref, [lane3 + (q * L * C + c)])
                if c:
                    iv = iv + c * V
                cidx_v[buf, c, pl.ds(q * L, L)] = iv
        rows_copy(0, buf).start()

    def fire_adds(buf):
        rows_copy(0, buf).wait()
        rows_copy(1, buf).start(add=True)
        rows_copy(2, buf).start(add=True)

    def combine(k, i, buf):
        rows_copy(1, buf).wait()
        rows_copy(2, buf).wait()
        for p in range(P):
            for q in range(D // (2 * L)):
                r = rows_v[buf, p, pl.ds(q * 2 * L, 2 * L)]
                lo, hi = plsc.unpack(r, format=plsc.PackFormat.INTERLEAVED)
                out_v[k, i, p, pl.ds(q * L, L)] = lo
                out_v[k, i, p, pl.ds(D // 2 + q * L, L)] = hi

    def outer(g0, carry):
        for k in range(2):
            g = g0 + k
            idx_copy(g, k).wait()
            @pl.when(g0 >= 2)
            def _():
                out_copy(g - 2, k).wait()

            def img(i, c2):
                @pl.when(i < G)
                def _():
                    build_and_fire0(k, i, lax.rem(i, 3))
                @pl.when((i >= 1) & (i <= G))
                def _():
                    fire_adds(lax.rem(i - 1, 3))
                @pl.when(i >= 2)
                def _():
                    combine(k, i - 2, lax.rem(i - 2, 3))
                return c2

            lax.fori_loop(0, G + 2, img, None)
            out_copy(g, k).start()
            @pl.when(g + 2 < NG)
            def _():
                idx_copy(g + 2, k).start()
        return carry

    lax.fori_loop(0, NG // 2, lambda t, c: outer(t * 2, c), None)

    out_copy(NG - 2, 0).wait()
    out_copy(NG - 1, 1).wait()


def kernel(inputs, embedding):
    b, h, w, c = inputs.shape
    eb = embedding.astype(jnp.bfloat16)
    inter = jnp.stack([eb[:, :D // 2], eb[:, D // 2:]],
                      axis=2).reshape(V3, D)
    inter = jnp.pad(inter, ((0, VP - V3), (0, 0)))
    idx = inputs.reshape(b, h * w * c).astype(jnp.int32)
    out = _bow_kernel(inter, idx)
    return jnp.transpose(out.reshape(b, h, w, D), (0, 3, 1, 2))


# speedup vs baseline: 2.8696x; 1.1730x over previous
import functools

import jax
import jax.numpy as jnp
from jax import lax
from jax.experimental import pallas as pl
from jax.experimental.pallas import tpu as pltpu
from jax.experimental.pallas import tpu_sc as plsc

V = 1000
V3 = 3 * V
VP = 3008
D = 128
B = 1024
P = 64
C = 3
L = 16

NC = 2
NS = 16
NW = NC * NS

BC = B // NW
IG = 8
NIG = BC // IG

_mesh = plsc.VectorSubcoreMesh(core_axis_name="c", subcore_axis_name="s")


@functools.partial(
    pl.kernel,
    out_type=jax.ShapeDtypeStruct((B, P, D), jnp.float32),
    mesh=_mesh,
    compiler_params=pltpu.CompilerParams(
        use_tc_tiling_on_sc=False, needs_layout_passes=False),
    scratch_types=[
        pltpu.VMEM_SHARED((VP, D), jnp.bfloat16),
        pltpu.VMEM((2, IG, P * C), jnp.int32),
        pltpu.VMEM((3, C, P), jnp.int32),
        pltpu.VMEM((3, P, D), jnp.bfloat16),
        pltpu.VMEM((3, P, D), jnp.float32),
        pltpu.SemaphoreType.DMA((2,)),
        pltpu.SemaphoreType.DMA((3,)),
        pltpu.SemaphoreType.DMA((3,)),
    ],
)
def _bow_kernel(tab_hbm, idx_hbm, out_hbm, tab_sh, idx_v, cidx_v, rows_v,
                out_v, idx_sem, rows_sem, out_sem):
    sid = lax.axis_index("s")
    wid = sid * NC + lax.axis_index("c")
    b0 = wid * BC

    def idx_copy(blk, k):
        return pltpu.make_async_copy(
            idx_hbm.at[pl.ds(b0 + blk * IG, IG)], idx_v.at[k],
            idx_sem.at[k])

    def out_copy(i, ob):
        return pltpu.make_async_copy(
            out_v.at[ob], out_hbm.at[b0 + i], out_sem.at[ob])

    def rows_copy(c, buf):
        return pltpu.make_async_copy(
            tab_sh.at[cidx_v.at[buf, c]], rows_v.at[buf],
            rows_sem.at[buf])

    idx_copy(0, 0).start()
    idx_copy(1, 1).start()
    rpt = VP // NS
    pltpu.sync_copy(tab_hbm.at[pl.ds(sid * rpt, rpt)],
                    tab_sh.at[pl.ds(sid * rpt, rpt)])
    plsc.subcore_barrier()

    lane3 = lax.iota(jnp.int32, 16) * C

    def step(i, carry):
        @pl.when(i < BC)
        def _():
            blk = i // IG
            @pl.when(lax.rem(i, IG) == 0)
            def _():
                idx_copy(blk, lax.rem(blk, 2)).wait()
            buf = lax.rem(i, 3)
            iref = idx_v.at[lax.rem(blk, 2), lax.rem(i, IG)]
            for c in range(C):
                for q in range(P // L):
                    iv = plsc.load_gather(iref, [lane3 + (q * L * C + c)])
                    if c:
                        iv = iv + c * V
                    cidx_v[buf, c, pl.ds(q * L, L)] = iv
            rows_copy(0, buf).start()
            @pl.when((lax.rem(i, IG) == IG - 1) & (blk + 2 < NIG))
            def _():
                idx_copy(blk + 2, lax.rem(blk, 2)).start()
        @pl.when((i >= 1) & (i <= BC))
        def _():
            buf = lax.rem(i - 1, 3)
            rows_copy(0, buf).wait()
            rows_copy(1, buf).start(add=True)
            rows_copy(2, buf).start(add=True)
        @pl.when(i >= 2)
        def _():
            j = i - 2
            buf = lax.rem(j, 3)
            @pl.when(j >= 3)
            def _():
                out_copy(j - 3, buf).wait()
            rows_copy(1, buf).wait()
            rows_copy(2, buf).wait()
            for p in range(P):
                for q in range(D // (2 * L)):
                    r = rows_v[buf, p, pl.ds(q * 2 * L, 2 * L)]
                    lo, hi = plsc.unpack(
                        r, format=plsc.PackFormat.INTERLEAVED)
                    out_v[buf, p, pl.ds(q * L, L)] = lo
                    out_v[buf, p, pl.ds(D // 2 + q * L, L)] = hi
            out_copy(j, buf).start()
        return carry

    lax.fori_loop(0, BC + 2, step, None)

    out_copy(BC - 3, lax.rem(BC - 3, 3)).wait()
    out_copy(BC - 2, lax.rem(BC - 2, 3)).wait()
    out_copy(BC - 1, lax.rem(BC - 1, 3)).wait()


def kernel(inputs, embedding):
    b, h, w, c = inputs.shape
    u = lax.bitcast_convert_type(embedding, jnp.uint32)
    rne = (u + 0x7FFF + ((u >> 16) & 1)) >> 16
    word = (rne[:, D // 2:] << 16) | rne[:, :D // 2]
    inter = lax.bitcast_convert_type(
        word.astype(jnp.uint32), jnp.bfloat16).reshape(V3, D)
    inter = jnp.pad(inter, ((0, VP - V3), (0, 0)))
    idx = inputs.reshape(b, h * w * c).astype(jnp.int32)
    out = _bow_kernel(inter, idx)
    return jnp.transpose(out.reshape(b, h, w, D), (0, 3, 1, 2))
